# baseline (speedup 1.0000x reference)
import jax
import jax.numpy as jnp
import numpy as np
from jax.experimental import pallas as pl
from jax.experimental.pallas import tpu as pltpu

PATCH_L = 512
PATCH_C = 64
NUM_PATCHES = 16
F_S = 100.0

_N = PATCH_L
_kn = (np.arange(_N)[:, None] * np.arange(_N)[None, :]) % _N
_ang = 2.0 * np.pi * _kn / _N
_DFT = np.concatenate([np.cos(_ang), -np.sin(_ang)], axis=1).astype(np.float32)


def _tf2x32(k0, k1, c0, c1):
    x0 = np.asarray(c0, np.uint32).copy()
    x1 = np.asarray(c1, np.uint32).copy()
    ks = [np.uint32(k0), np.uint32(k1),
          np.uint32(np.uint32(k0) ^ np.uint32(k1) ^ np.uint32(0x1BD11BDA))]
    rots = [(13, 15, 26, 6), (17, 29, 16, 24)]
    x0 = (x0 + ks[0]).astype(np.uint32)
    x1 = (x1 + ks[1]).astype(np.uint32)
    for i in range(5):
        for r in rots[i % 2]:
            x0 = (x0 + x1).astype(np.uint32)
            x1 = ((x1 << np.uint32(r)) | (x1 >> np.uint32(32 - r)))
            x1 = (x1 ^ x0).astype(np.uint32)
        x0 = (x0 + ks[(i + 1) % 3]).astype(np.uint32)
        x1 = (x1 + ks[(i + 2) % 3] + np.uint32(i + 1)).astype(np.uint32)
    return x0, x1


def _tf_split(key):
    b1, b2 = _tf2x32(key[0], key[1], np.zeros(2, np.uint32),
                     np.arange(2, dtype=np.uint32))
    return (b1[0], b2[0]), (b1[1], b2[1])


def _tf_rbits(key, size):
    b1, b2 = _tf2x32(key[0], key[1], np.zeros(size, np.uint32),
                     np.arange(size, dtype=np.uint32))
    return (b1 ^ b2).astype(np.uint32)


def _tf_randint(key, shape, maxval):
    size = int(np.prod(shape))
    k1, k2 = _tf_split(key)
    hi, lo = _tf_rbits(k1, size), _tf_rbits(k2, size)
    span = np.uint32(maxval)
    mult = np.uint32(((2 ** 16) % maxval) ** 2 % maxval)
    off = ((hi % span) * mult + lo % span) % span
    return off.astype(np.int32).reshape(shape)


def _patch_starts(B, L, C):
    kL, kC = _tf_split((np.uint32(0), np.uint32(42)))
    start_L = _tf_randint(kL, (B, NUM_PATCHES), L - PATCH_L + 1)
    start_C = _tf_randint(kC, (B, NUM_PATCHES), C - PATCH_C + 1)
    return start_L, start_C


def _fft_body(sl_ref, sc_ref, x_ref, dft_ref, out_ref):
    b = pl.program_id(0)
    C = x_ref.shape[2]
    cols = []
    for p in range(NUM_PATCHES):
        i = b * NUM_PATCHES + p
        sl = sl_ref[i]
        sc = sc_ref[i]
        xs = x_ref[0, pl.ds(sl, PATCH_L), :]
        cols.append(pltpu.roll(xs, C - sc, axis=1)[:, :PATCH_C])
    patches = jnp.concatenate(cols, axis=1)
    res_t = jax.lax.dot_general(
        patches, dft_ref[...], (((0,), (0,)), ((), ())),
        preferred_element_type=jnp.float32,
    )
    for p in range(NUM_PATCHES):
        rows = res_t[p * PATCH_C:(p + 1) * PATCH_C]
        out_ref[0, 0, p] = rows[:, :PATCH_L]
        out_ref[1, 0, p] = rows[:, PATCH_L:]


def kernel(x):
    B, L, C = x.shape
    start_L, start_C = _patch_starts(B, L, C)
    sl_flat = start_L.reshape(-1).astype(np.int32)
    sc_flat = start_C.reshape(-1).astype(np.int32)
    dft = jnp.asarray(_DFT)

    grid_spec = pltpu.PrefetchScalarGridSpec(
        num_scalar_prefetch=2,
        grid=(B,),
        in_specs=[
            pl.BlockSpec((1, L, C), lambda b, *_: (b, 0, 0)),
            pl.BlockSpec((PATCH_L, 2 * PATCH_L), lambda b, *_: (0, 0)),
        ],
        out_specs=[
            pl.BlockSpec((2, 1, NUM_PATCHES, PATCH_C, PATCH_L),
                         lambda b, *_: (0, b, 0, 0, 0)),
        ],
    )
    out = pl.pallas_call(
        _fft_body,
        grid_spec=grid_spec,
        out_shape=[
            jax.ShapeDtypeStruct(
                (2, B, NUM_PATCHES, PATCH_C, PATCH_L), jnp.float32),
        ],
    )(sl_flat, sc_flat, x, dft)[0]

    patches_fft = out.transpose(1, 2, 4, 3, 0)
    t = jnp.broadcast_to(
        (jnp.arange(L, dtype=jnp.float32) * (1.0 / F_S))[None, :], (B, L)
    )
    return (patches_fft, t)


# speedup vs baseline: 1.0526x; 1.0526x over previous
import jax
import jax.numpy as jnp
import numpy as np
from jax.experimental import pallas as pl
from jax.experimental.pallas import tpu as pltpu

PATCH_L = 512
PATCH_C = 64
NUM_PATCHES = 16
F_S = 100.0

_N = PATCH_L
_kn = (np.arange(_N)[:, None] * np.arange(_N)[None, :]) % _N
_ang = 2.0 * np.pi * _kn / _N
_DFT = np.concatenate([np.cos(_ang), -np.sin(_ang)], axis=1).astype(np.float32)


def _tf2x32(k0, k1, c0, c1):
    x0 = np.asarray(c0, np.uint32).copy()
    x1 = np.asarray(c1, np.uint32).copy()
    ks = [np.uint32(k0), np.uint32(k1),
          np.uint32(np.uint32(k0) ^ np.uint32(k1) ^ np.uint32(0x1BD11BDA))]
    rots = [(13, 15, 26, 6), (17, 29, 16, 24)]
    x0 = (x0 + ks[0]).astype(np.uint32)
    x1 = (x1 + ks[1]).astype(np.uint32)
    for i in range(5):
        for r in rots[i % 2]:
            x0 = (x0 + x1).astype(np.uint32)
            x1 = ((x1 << np.uint32(r)) | (x1 >> np.uint32(32 - r)))
            x1 = (x1 ^ x0).astype(np.uint32)
        x0 = (x0 + ks[(i + 1) % 3]).astype(np.uint32)
        x1 = (x1 + ks[(i + 2) % 3] + np.uint32(i + 1)).astype(np.uint32)
    return x0, x1


def _tf_split(key):
    b1, b2 = _tf2x32(key[0], key[1], np.zeros(2, np.uint32),
                     np.arange(2, dtype=np.uint32))
    return (b1[0], b2[0]), (b1[1], b2[1])


def _tf_rbits(key, size):
    b1, b2 = _tf2x32(key[0], key[1], np.zeros(size, np.uint32),
                     np.arange(size, dtype=np.uint32))
    return (b1 ^ b2).astype(np.uint32)


def _tf_randint(key, shape, maxval):
    size = int(np.prod(shape))
    k1, k2 = _tf_split(key)
    hi, lo = _tf_rbits(k1, size), _tf_rbits(k2, size)
    span = np.uint32(maxval)
    mult = np.uint32(((2 ** 16) % maxval) ** 2 % maxval)
    off = ((hi % span) * mult + lo % span) % span
    return off.astype(np.int32).reshape(shape)


def _patch_starts(B, L, C):
    kL, kC = _tf_split((np.uint32(0), np.uint32(42)))
    start_L = _tf_randint(kL, (B, NUM_PATCHES), L - PATCH_L + 1)
    start_C = _tf_randint(kC, (B, NUM_PATCHES), C - PATCH_C + 1)
    return start_L, start_C


BGROUP = 2


def _fft_body(sl_ref, sc_ref, x_ref, dft_ref, out_ref):
    g = pl.program_id(0)
    C = x_ref.shape[2]
    cols = []
    for bb in range(BGROUP):
        for p in range(NUM_PATCHES):
            i = (g * BGROUP + bb) * NUM_PATCHES + p
            sl = sl_ref[i]
            sc = sc_ref[i]
            xs = x_ref[bb, pl.ds(sl, PATCH_L), :]
            cols.append(pltpu.roll(xs, C - sc, axis=1)[:, :PATCH_C])
    patches = jnp.concatenate(cols, axis=1)
    res_t = jax.lax.dot_general(
        patches, dft_ref[...], (((0,), (0,)), ((), ())),
        preferred_element_type=jnp.float32,
    )
    for bb in range(BGROUP):
        for p in range(NUM_PATCHES):
            r0 = (bb * NUM_PATCHES + p) * PATCH_C
            rows = res_t[r0:r0 + PATCH_C]
            out_ref[0, bb, p] = rows[:, :PATCH_L]
            out_ref[1, bb, p] = rows[:, PATCH_L:]


def kernel(x):
    B, L, C = x.shape
    start_L, start_C = _patch_starts(B, L, C)
    sl_flat = start_L.reshape(-1).astype(np.int32)
    sc_flat = start_C.reshape(-1).astype(np.int32)
    dft = jnp.asarray(_DFT)

    grid_spec = pltpu.PrefetchScalarGridSpec(
        num_scalar_prefetch=2,
        grid=(B // BGROUP,),
        in_specs=[
            pl.BlockSpec((BGROUP, L, C), lambda g, *_: (g, 0, 0)),
            pl.BlockSpec((PATCH_L, 2 * PATCH_L), lambda g, *_: (0, 0)),
        ],
        out_specs=[
            pl.BlockSpec((2, BGROUP, NUM_PATCHES, PATCH_C, PATCH_L),
                         lambda g, *_: (0, g, 0, 0, 0)),
        ],
    )
    out = pl.pallas_call(
        _fft_body,
        grid_spec=grid_spec,
        out_shape=[
            jax.ShapeDtypeStruct(
                (2, B, NUM_PATCHES, PATCH_C, PATCH_L), jnp.float32),
        ],
    )(sl_flat, sc_flat, x, dft)[0]

    patches_fft = out.transpose(1, 2, 4, 3, 0)
    t = jnp.broadcast_to(
        (jnp.arange(L, dtype=jnp.float32) * (1.0 / F_S))[None, :], (B, L)
    )
    return (patches_fft, t)
